# baseline (speedup 1.0000x reference)
import functools

import jax
import jax.numpy as jnp
from jax import lax
from jax.experimental import pallas as pl
from jax.experimental.pallas import tpu as pltpu
from jax.experimental.pallas import tpu_sc as plsc

N = 10000
E = 320000
D = 128
ET = 4
B = 16
STEPS = 8
HID = 256
RES = 768

NC = 2
NS = 16
NW = NC * NS
CSZ = 128
CH = 80
EPW = CH * CSZ
E_PAD = NW * EPW
NB = 2
RPT = 640
NPAD = NS * RPT
LAST = N - (NS - 1) * RPT


def _sc_aggregate(y2, gidx_p, dst_p, zrow):
    mesh = plsc.VectorSubcoreMesh(
        core_axis_name="c", subcore_axis_name="s", num_cores=NC, num_subcores=NS
    )

    @functools.partial(
        pl.kernel,
        out_type=jax.ShapeDtypeStruct((NC, N, D), jnp.float32),
        mesh=mesh,
        scratch_types=[
            pltpu.VMEM((CH, CSZ), jnp.int32),
            pltpu.VMEM((NB * 8, CSZ), jnp.int32),
            pltpu.VMEM((NB, CSZ, D), jnp.float32),
            pltpu.VMEM_SHARED((NPAD, D), jnp.float32),
            pltpu.SemaphoreType.DMA,
            pltpu.SemaphoreType.DMA,
            pltpu.SemaphoreType.DMA,
            pltpu.SemaphoreType.DMA,
            pltpu.SemaphoreType.DMA,
            pltpu.SemaphoreType.DMA,
        ],
    )
    def body(y_hbm, gidx_hbm, dst_hbm, z_hbm, out_hbm,
             gidx_v, didx_v, rows_v, m_sh, gs0, gs1, ds0, ds1, ss0, ss1):
        gsem = [gs0, gs1]
        dsem = [ds0, ds1]
        ssem = [ss0, ss1]
        c = lax.axis_index("c")
        s = lax.axis_index("s")
        wid = c * NS + s

        pltpu.sync_copy(z_hbm, m_sh.at[pl.ds(s * RPT, RPT)])

        pltpu.sync_copy(gidx_hbm.at[wid], gidx_v)

        plsc.subcore_barrier()

        dchunks = dst_hbm.at[wid]

        def issue(j, b):
            pltpu.async_copy(
                y_hbm.at[gidx_v.at[j]], rows_v.at[b], gsem[b]
            )
            pltpu.async_copy(dchunks.at[j], didx_v.at[b * 8], dsem[b])

        def wait(j, b):
            pltpu.make_async_copy(
                y_hbm.at[gidx_v.at[j]], rows_v.at[b], gsem[b]
            ).wait()
            pltpu.make_async_copy(dchunks.at[j], didx_v.at[b * 8], dsem[b]).wait()

        for b in range(NB):
            issue(b, b)

        @pl.loop(0, CH, step=NB)
        def _chunk(j0):
            for b in range(NB):
                wait(j0 + b, b)
                pltpu.async_copy(
                    rows_v.at[b], m_sh.at[didx_v.at[b * 8]], ssem[b], add=True
                )
            for b in range(NB):
                pltpu.make_async_copy(
                    rows_v.at[b], m_sh.at[didx_v.at[b * 8]], ssem[b]
                ).wait()

                @pl.when(j0 + b + NB < CH)
                def _():
                    issue(j0 + b + NB, b)

        plsc.subcore_barrier()

        @pl.when(s < NS - 1)
        def _():
            pltpu.sync_copy(
                m_sh.at[pl.ds(s * RPT, RPT)], out_hbm.at[c].at[pl.ds(s * RPT, RPT)]
            )

        @pl.when(s == NS - 1)
        def _():
            pltpu.sync_copy(
                m_sh.at[pl.ds((NS - 1) * RPT, LAST)],
                out_hbm.at[c].at[pl.ds((NS - 1) * RPT, LAST)],
            )

    return body(y2, gidx_p, dst_p, zrow)


BLK = 1000
GRID = N // BLK


def _y_blocks(hn, wcat_ref, be_ref, y_ref):
    for t in range(ET):
        y_ref[t] = (
            jnp.dot(hn, wcat_ref[t], preferred_element_type=jnp.float32)
            + be_ref[t]
        )


_Y_SPECS = [
    pl.BlockSpec((ET, D, D), lambda i: (0, 0, 0)),
    pl.BlockSpec((ET, 1, D), lambda i: (0, 0, 0)),
]
_Y_OUT_SPEC = pl.BlockSpec((ET, BLK, D), lambda i: (0, i, 0))
_Y_OUT_SHAPE = jax.ShapeDtypeStruct((ET, N, D), jnp.float32)


def _tc_y(h, wcat, be_r):
    def body(h_ref, wcat_ref, be_ref, y_ref):
        _y_blocks(h_ref[...], wcat_ref, be_ref, y_ref)

    return pl.pallas_call(
        body,
        grid=(GRID,),
        in_specs=[pl.BlockSpec((BLK, D), lambda i: (i, 0))] + _Y_SPECS,
        out_specs=_Y_OUT_SPEC,
        out_shape=_Y_OUT_SHAPE,
    )(h, wcat, be_r)


def _gru_block(parts_ref, h, wih_ref, whh_ref, bih_ref, bhh_ref):
    m = parts_ref[0] + parts_ref[1]
    gi = jnp.dot(m, wih_ref[...], preferred_element_type=jnp.float32) + bih_ref[...]
    gh = jnp.dot(h, whh_ref[...], preferred_element_type=jnp.float32) + bhh_ref[...]
    r = jax.nn.sigmoid(gi[:, :D] + gh[:, :D])
    z = jax.nn.sigmoid(gi[:, D:2 * D] + gh[:, D:2 * D])
    n = jnp.tanh(gi[:, 2 * D:] + r * gh[:, 2 * D:])
    return (1.0 - z) * n + z * h


_GRU_SPECS = [
    pl.BlockSpec((NC, BLK, D), lambda i: (0, i, 0)),
    pl.BlockSpec((BLK, D), lambda i: (i, 0)),
    pl.BlockSpec((D, 3 * D), lambda i: (0, 0)),
    pl.BlockSpec((D, 3 * D), lambda i: (0, 0)),
    pl.BlockSpec((1, 3 * D), lambda i: (0, 0)),
    pl.BlockSpec((1, 3 * D), lambda i: (0, 0)),
]


def _tc_gru_y(parts, h, wih_t, whh_t, bih_r, bhh_r, wcat, be_r):
    def body(parts_ref, h_ref, wih_ref, whh_ref, bih_ref, bhh_ref,
             wcat_ref, be_ref, h_out_ref, y_ref):
        hn = _gru_block(parts_ref, h_ref[...], wih_ref, whh_ref, bih_ref, bhh_ref)
        h_out_ref[...] = hn
        _y_blocks(hn, wcat_ref, be_ref, y_ref)

    return pl.pallas_call(
        body,
        grid=(GRID,),
        in_specs=_GRU_SPECS + _Y_SPECS,
        out_specs=[pl.BlockSpec((BLK, D), lambda i: (i, 0)), _Y_OUT_SPEC],
        out_shape=[
            jax.ShapeDtypeStruct((N, D), jnp.float32),
            _Y_OUT_SHAPE,
        ],
    )(parts, h, wih_t, whh_t, bih_r, bhh_r, wcat, be_r)


def _tc_final(parts, h, res, gids3, wih_t, whh_t, bih_r, bhh_r,
              w1t, b1_r, w2t, b2_r):
    def body(parts_ref, h_ref, res_ref, gid_ref, wih_ref, whh_ref,
             bih_ref, bhh_ref, w1t_ref, b1_ref, w2t_ref, b2_ref,
             out_ref, sums_ref, cnt_ref):
        i = pl.program_id(0)

        @pl.when(i == 0)
        def _():
            sums_ref[...] = jnp.zeros_like(sums_ref)
            cnt_ref[...] = jnp.zeros_like(cnt_ref)

        hn = _gru_block(parts_ref, h_ref[...], wih_ref, whh_ref, bih_ref, bhh_ref)

        ids = gid_ref[0, 0, :]
        oh = (
            lax.broadcasted_iota(jnp.int32, (B, BLK), 0) == ids[None, :]
        ).astype(jnp.float32)
        sums_ref[:, :D] += jnp.dot(oh, hn, preferred_element_type=jnp.float32)
        sums_ref[:, D:] += jnp.dot(
            oh, res_ref[...], preferred_element_type=jnp.float32
        )
        cnt_ref[...] += jnp.dot(
            oh, jnp.ones((BLK, D), jnp.float32), preferred_element_type=jnp.float32
        )

        @pl.when(i == pl.num_programs(0) - 1)
        def _():
            cnt = jnp.maximum(cnt_ref[...], 1.0)
            g = sums_ref[...] / jnp.concatenate([cnt] * ((D + RES) // D), axis=1)
            h1 = jax.nn.relu(
                jnp.dot(g, w1t_ref[...], preferred_element_type=jnp.float32)
                + b1_ref[...]
            )
            logit = (
                jnp.dot(h1, w2t_ref[...], preferred_element_type=jnp.float32)
                + b2_ref[...]
            )
            out_ref[...] = jax.nn.sigmoid(logit)

    return pl.pallas_call(
        body,
        grid=(GRID,),
        in_specs=_GRU_SPECS[:2] + [
            pl.BlockSpec((BLK, RES), lambda i: (i, 0)),
            pl.BlockSpec((1, 1, BLK), lambda i: (i, 0, 0)),
        ] + _GRU_SPECS[2:] + [
            pl.BlockSpec((D + RES, HID), lambda i: (0, 0)),
            pl.BlockSpec((1, HID), lambda i: (0, 0)),
            pl.BlockSpec((HID, 1), lambda i: (0, 0)),
            pl.BlockSpec((1, 1), lambda i: (0, 0)),
        ],
        out_specs=pl.BlockSpec((B, 1), lambda i: (0, 0)),
        out_shape=jax.ShapeDtypeStruct((B, 1), jnp.float32),
        scratch_shapes=[
            pltpu.VMEM((B, D + RES), jnp.float32),
            pltpu.VMEM((B, D), jnp.float32),
        ],
    )(parts, h, res, gids3, wih_t, whh_t, bih_r, bhh_r, w1t, b1_r, w2t, b2_r)


def kernel(features, edge_index, edge_types, graph_ids, We, be,
           W_ih, W_hh, b_ih, b_hh, W1, b1, W2, b2):
    x = features[:, :D]
    res = features[:, D:]

    src = edge_index[0].astype(jnp.int32)
    dst = edge_index[1].astype(jnp.int32)
    et = edge_types.astype(jnp.int32)
    pad = E_PAD - E
    gidx = et * N + src
    pad_g = jnp.arange(pad, dtype=jnp.int32) % (ET * N)
    gidx_p = jnp.concatenate([gidx, pad_g]).reshape(NW, CH, CSZ)
    pad_d = N + jnp.arange(pad, dtype=jnp.int32) % (NPAD - N)
    dst_p = jnp.concatenate([dst, pad_d]).reshape(NW, CH, CSZ)
    zrow = jnp.zeros((RPT, D), jnp.float32)

    wcat = jnp.transpose(We, (0, 2, 1))
    be_r = be[:, None, :]
    wih_t = W_ih.T
    whh_t = W_hh.T
    bih_r = b_ih[None, :]
    bhh_r = b_hh[None, :]
    w1t = W1.T
    b1_r = b1[None, :]
    w2t = W2.T
    b2_r = b2[None, :]
    gids3 = graph_ids.astype(jnp.int32).reshape(GRID, 1, BLK)

    h = x
    y = _tc_y(h, wcat, be_r)
    for step in range(STEPS):
        parts = _sc_aggregate(y.reshape(ET * N, D), gidx_p, dst_p, zrow)
        if step < STEPS - 1:
            h, y = _tc_gru_y(parts, h, wih_t, whh_t, bih_r, bhh_r, wcat, be_r)
        else:
            out = _tc_final(parts, h, res, gids3, wih_t, whh_t,
                            bih_r, bhh_r, w1t, b1_r, w2t, b2_r)
    return out


# speedup vs baseline: 1.2712x; 1.2712x over previous
import functools

import jax
import jax.numpy as jnp
from jax import lax
from jax.experimental import pallas as pl
from jax.experimental.pallas import tpu as pltpu
from jax.experimental.pallas import tpu_sc as plsc

N = 10000
E = 320000
D = 128
ET = 4
B = 16
STEPS = 8
HID = 256
RES = 768

NC = 2
NS = 16
NW = NC * NS
CSZ = 128
CH = 80
EPW = CH * CSZ
E_PAD = NW * EPW
NB = 2
RPT = 640
NPAD = NS * RPT
LAST = N - (NS - 1) * RPT


def _sc_aggregate(y2, gidx_p, dst_p, zrow):
    mesh = plsc.VectorSubcoreMesh(
        core_axis_name="c", subcore_axis_name="s", num_cores=NC, num_subcores=NS
    )

    @functools.partial(
        pl.kernel,
        out_type=jax.ShapeDtypeStruct((NC, N, D), jnp.float32),
        mesh=mesh,
        scratch_types=[
            pltpu.VMEM((CH, CSZ), jnp.int32),
            pltpu.VMEM((NB * 8, CSZ), jnp.int32),
            pltpu.VMEM((NB, CSZ, D), jnp.float32),
            pltpu.VMEM_SHARED((NPAD, D), jnp.float32),
            pltpu.SemaphoreType.DMA,
            pltpu.SemaphoreType.DMA,
            pltpu.SemaphoreType.DMA,
            pltpu.SemaphoreType.DMA,
        ],
    )
    def body(y_hbm, gidx_hbm, dst_hbm, z_hbm, out_hbm,
             gidx_v, didx_v, rows_v, m_sh, gs0, gs1, ds0, ds1):
        gsem = [gs0, gs1]
        dsem = [ds0, ds1]
        c = lax.axis_index("c")
        s = lax.axis_index("s")
        wid = c * NS + s

        pltpu.sync_copy(z_hbm, m_sh.at[pl.ds(s * RPT, RPT)])

        pltpu.sync_copy(gidx_hbm.at[wid], gidx_v)

        plsc.subcore_barrier()

        dchunks = dst_hbm.at[wid]

        def issue(j, b):
            pltpu.async_copy(
                y_hbm.at[gidx_v.at[j]], rows_v.at[b], gsem[b]
            )
            pltpu.async_copy(dchunks.at[j], didx_v.at[b * 8], dsem[b])

        def wait(j, b):
            pltpu.make_async_copy(
                y_hbm.at[gidx_v.at[j]], rows_v.at[b], gsem[b]
            ).wait()
            pltpu.make_async_copy(dchunks.at[j], didx_v.at[b * 8], dsem[b]).wait()

        for b in range(NB):
            issue(b, b)

        @pl.loop(0, CH, step=NB)
        def _chunk(j0):
            for b in range(NB):
                j = j0 + b
                wait(j, b)
                pltpu.sync_copy(
                    rows_v.at[b], m_sh.at[didx_v.at[b * 8]], add=True
                )

                @pl.when(j + NB < CH)
                def _():
                    issue(j + NB, b)

        plsc.subcore_barrier()

        @pl.when(s < NS - 1)
        def _():
            pltpu.sync_copy(
                m_sh.at[pl.ds(s * RPT, RPT)], out_hbm.at[c].at[pl.ds(s * RPT, RPT)]
            )

        @pl.when(s == NS - 1)
        def _():
            pltpu.sync_copy(
                m_sh.at[pl.ds((NS - 1) * RPT, LAST)],
                out_hbm.at[c].at[pl.ds((NS - 1) * RPT, LAST)],
            )

    return body(y2, gidx_p, dst_p, zrow)


BLK = 1000
GRID = N // BLK


def _y_blocks(hn, wcat_ref, be_ref, y_ref):
    for t in range(ET):
        y_ref[t] = (
            jnp.dot(hn, wcat_ref[t], preferred_element_type=jnp.float32)
            + be_ref[t]
        )


_Y_SPECS = [
    pl.BlockSpec((ET, D, D), lambda i: (0, 0, 0)),
    pl.BlockSpec((ET, 1, D), lambda i: (0, 0, 0)),
]
_Y_OUT_SPEC = pl.BlockSpec((ET, BLK, D), lambda i: (0, i, 0))
_Y_OUT_SHAPE = jax.ShapeDtypeStruct((ET, N, D), jnp.float32)


def _tc_y(h, wcat, be_r):
    def body(h_ref, wcat_ref, be_ref, y_ref):
        _y_blocks(h_ref[...], wcat_ref, be_ref, y_ref)

    return pl.pallas_call(
        body,
        grid=(GRID,),
        in_specs=[pl.BlockSpec((BLK, D), lambda i: (i, 0))] + _Y_SPECS,
        out_specs=_Y_OUT_SPEC,
        out_shape=_Y_OUT_SHAPE,
    )(h, wcat, be_r)


def _gru_block(parts_ref, h, wih_ref, whh_ref, bih_ref, bhh_ref):
    m = parts_ref[0] + parts_ref[1]
    gi = jnp.dot(m, wih_ref[...], preferred_element_type=jnp.float32) + bih_ref[...]
    gh = jnp.dot(h, whh_ref[...], preferred_element_type=jnp.float32) + bhh_ref[...]
    r = jax.nn.sigmoid(gi[:, :D] + gh[:, :D])
    z = jax.nn.sigmoid(gi[:, D:2 * D] + gh[:, D:2 * D])
    n = jnp.tanh(gi[:, 2 * D:] + r * gh[:, 2 * D:])
    return (1.0 - z) * n + z * h


_GRU_SPECS = [
    pl.BlockSpec((NC, BLK, D), lambda i: (0, i, 0)),
    pl.BlockSpec((BLK, D), lambda i: (i, 0)),
    pl.BlockSpec((D, 3 * D), lambda i: (0, 0)),
    pl.BlockSpec((D, 3 * D), lambda i: (0, 0)),
    pl.BlockSpec((1, 3 * D), lambda i: (0, 0)),
    pl.BlockSpec((1, 3 * D), lambda i: (0, 0)),
]


def _tc_gru_y(parts, h, wih_t, whh_t, bih_r, bhh_r, wcat, be_r):
    def body(parts_ref, h_ref, wih_ref, whh_ref, bih_ref, bhh_ref,
             wcat_ref, be_ref, h_out_ref, y_ref):
        hn = _gru_block(parts_ref, h_ref[...], wih_ref, whh_ref, bih_ref, bhh_ref)
        h_out_ref[...] = hn
        _y_blocks(hn, wcat_ref, be_ref, y_ref)

    return pl.pallas_call(
        body,
        grid=(GRID,),
        in_specs=_GRU_SPECS + _Y_SPECS,
        out_specs=[pl.BlockSpec((BLK, D), lambda i: (i, 0)), _Y_OUT_SPEC],
        out_shape=[
            jax.ShapeDtypeStruct((N, D), jnp.float32),
            _Y_OUT_SHAPE,
        ],
    )(parts, h, wih_t, whh_t, bih_r, bhh_r, wcat, be_r)


def _tc_final(parts, h, res, gids3, wih_t, whh_t, bih_r, bhh_r,
              w1t, b1_r, w2t, b2_r):
    def body(parts_ref, h_ref, res_ref, gid_ref, wih_ref, whh_ref,
             bih_ref, bhh_ref, w1t_ref, b1_ref, w2t_ref, b2_ref,
             out_ref, sums_ref, cnt_ref):
        i = pl.program_id(0)

        @pl.when(i == 0)
        def _():
            sums_ref[...] = jnp.zeros_like(sums_ref)
            cnt_ref[...] = jnp.zeros_like(cnt_ref)

        hn = _gru_block(parts_ref, h_ref[...], wih_ref, whh_ref, bih_ref, bhh_ref)

        ids = gid_ref[0, 0, :]
        oh = (
            lax.broadcasted_iota(jnp.int32, (B, BLK), 0) == ids[None, :]
        ).astype(jnp.float32)
        sums_ref[:, :D] += jnp.dot(oh, hn, preferred_element_type=jnp.float32)
        sums_ref[:, D:] += jnp.dot(
            oh, res_ref[...], preferred_element_type=jnp.float32
        )
        cnt_ref[...] += jnp.dot(
            oh, jnp.ones((BLK, D), jnp.float32), preferred_element_type=jnp.float32
        )

        @pl.when(i == pl.num_programs(0) - 1)
        def _():
            cnt = jnp.maximum(cnt_ref[...], 1.0)
            g = sums_ref[...] / jnp.concatenate([cnt] * ((D + RES) // D), axis=1)
            h1 = jax.nn.relu(
                jnp.dot(g, w1t_ref[...], preferred_element_type=jnp.float32)
                + b1_ref[...]
            )
            logit = (
                jnp.dot(h1, w2t_ref[...], preferred_element_type=jnp.float32)
                + b2_ref[...]
            )
            out_ref[...] = jax.nn.sigmoid(logit)

    return pl.pallas_call(
        body,
        grid=(GRID,),
        in_specs=_GRU_SPECS[:2] + [
            pl.BlockSpec((BLK, RES), lambda i: (i, 0)),
            pl.BlockSpec((1, 1, BLK), lambda i: (i, 0, 0)),
        ] + _GRU_SPECS[2:] + [
            pl.BlockSpec((D + RES, HID), lambda i: (0, 0)),
            pl.BlockSpec((1, HID), lambda i: (0, 0)),
            pl.BlockSpec((HID, 1), lambda i: (0, 0)),
            pl.BlockSpec((1, 1), lambda i: (0, 0)),
        ],
        out_specs=pl.BlockSpec((B, 1), lambda i: (0, 0)),
        out_shape=jax.ShapeDtypeStruct((B, 1), jnp.float32),
        scratch_shapes=[
            pltpu.VMEM((B, D + RES), jnp.float32),
            pltpu.VMEM((B, D), jnp.float32),
        ],
    )(parts, h, res, gids3, wih_t, whh_t, bih_r, bhh_r, w1t, b1_r, w2t, b2_r)


def kernel(features, edge_index, edge_types, graph_ids, We, be,
           W_ih, W_hh, b_ih, b_hh, W1, b1, W2, b2):
    x = features[:, :D]
    res = features[:, D:]

    src = edge_index[0].astype(jnp.int32)
    dst = edge_index[1].astype(jnp.int32)
    et = edge_types.astype(jnp.int32)
    pad = E_PAD - E
    gidx = et * N + src
    pad_g = jnp.arange(pad, dtype=jnp.int32) % (ET * N)
    gidx_p = jnp.concatenate([gidx, pad_g]).reshape(NW, CH, CSZ)
    pad_d = N + jnp.arange(pad, dtype=jnp.int32) % (NPAD - N)
    dst_p = jnp.concatenate([dst, pad_d]).reshape(NW, CH, CSZ)
    zrow = jnp.zeros((RPT, D), jnp.float32)

    wcat = jnp.transpose(We, (0, 2, 1))
    be_r = be[:, None, :]
    wih_t = W_ih.T
    whh_t = W_hh.T
    bih_r = b_ih[None, :]
    bhh_r = b_hh[None, :]
    w1t = W1.T
    b1_r = b1[None, :]
    w2t = W2.T
    b2_r = b2[None, :]
    gids3 = graph_ids.astype(jnp.int32).reshape(GRID, 1, BLK)

    h = x
    y = _tc_y(h, wcat, be_r)
    for step in range(STEPS):
        parts = _sc_aggregate(y.reshape(ET * N, D), gidx_p, dst_p, zrow)
        if step < STEPS - 1:
            h, y = _tc_gru_y(parts, h, wih_t, whh_t, bih_r, bhh_r, wcat, be_r)
        else:
            out = _tc_final(parts, h, res, gids3, wih_t, whh_t,
                            bih_r, bhh_r, w1t, b1_r, w2t, b2_r)
    return out


# speedup vs baseline: 1.2811x; 1.0078x over previous
import functools

import jax
import jax.numpy as jnp
from jax import lax
from jax.experimental import pallas as pl
from jax.experimental.pallas import tpu as pltpu
from jax.experimental.pallas import tpu_sc as plsc

N = 10000
E = 320000
D = 128
ET = 4
B = 16
STEPS = 8
HID = 256
RES = 768

NC = 2
NS = 16
NW = NC * NS
CSZ = 128
CH = 80
EPW = CH * CSZ
E_PAD = NW * EPW
NB = 2
RPT = 640
NPAD = NS * RPT
LAST = N - (NS - 1) * RPT


def _sc_aggregate(y2, gidx_p, dst_p, zrow):
    mesh = plsc.VectorSubcoreMesh(
        core_axis_name="c", subcore_axis_name="s", num_cores=NC, num_subcores=NS
    )

    @functools.partial(
        pl.kernel,
        out_type=jax.ShapeDtypeStruct((NC, N, D), jnp.float32),
        mesh=mesh,
        scratch_types=[
            pltpu.VMEM((CH, CSZ), jnp.int32),
            pltpu.VMEM((NB * 8, CSZ), jnp.int32),
            pltpu.VMEM((NB, CSZ, D), jnp.float32),
            pltpu.VMEM_SHARED((NPAD, D), jnp.float32),
            pltpu.SemaphoreType.DMA,
            pltpu.SemaphoreType.DMA,
            pltpu.SemaphoreType.DMA,
            pltpu.SemaphoreType.DMA,
        ],
    )
    def body(y_hbm, gidx_hbm, dst_hbm, z_hbm, out_hbm,
             gidx_v, didx_v, rows_v, m_sh, gs0, gs1, ds0, ds1):
        gsem = [gs0, gs1]
        dsem = [ds0, ds1]
        c = lax.axis_index("c")
        s = lax.axis_index("s")
        wid = c * NS + s

        dchunks = dst_hbm.at[wid]

        def issue(j, b):
            pltpu.async_copy(
                y_hbm.at[gidx_v.at[j]], rows_v.at[b], gsem[b]
            )
            pltpu.async_copy(dchunks.at[j], didx_v.at[b * 8], dsem[b])

        def wait(j, b):
            pltpu.make_async_copy(
                y_hbm.at[gidx_v.at[j]], rows_v.at[b], gsem[b]
            ).wait()
            pltpu.make_async_copy(dchunks.at[j], didx_v.at[b * 8], dsem[b]).wait()

        pltpu.sync_copy(gidx_hbm.at[wid], gidx_v)
        for b in range(NB):
            issue(b, b)

        pltpu.sync_copy(z_hbm, m_sh.at[pl.ds(s * RPT, RPT)])
        plsc.subcore_barrier()

        @pl.loop(0, CH, step=NB)
        def _chunk(j0):
            for b in range(NB):
                j = j0 + b
                wait(j, b)
                pltpu.sync_copy(
                    rows_v.at[b], m_sh.at[didx_v.at[b * 8]], add=True
                )

                @pl.when(j + NB < CH)
                def _():
                    issue(j + NB, b)

        plsc.subcore_barrier()

        @pl.when(s < NS - 1)
        def _():
            pltpu.sync_copy(
                m_sh.at[pl.ds(s * RPT, RPT)], out_hbm.at[c].at[pl.ds(s * RPT, RPT)]
            )

        @pl.when(s == NS - 1)
        def _():
            pltpu.sync_copy(
                m_sh.at[pl.ds((NS - 1) * RPT, LAST)],
                out_hbm.at[c].at[pl.ds((NS - 1) * RPT, LAST)],
            )

    return body(y2, gidx_p, dst_p, zrow)


BLK = 1000
GRID = N // BLK


def _y_blocks(hn, wcat_ref, be_ref, y_ref):
    for t in range(ET):
        y_ref[t] = (
            jnp.dot(hn, wcat_ref[t], preferred_element_type=jnp.float32)
            + be_ref[t]
        )


_Y_SPECS = [
    pl.BlockSpec((ET, D, D), lambda i: (0, 0, 0)),
    pl.BlockSpec((ET, 1, D), lambda i: (0, 0, 0)),
]
_Y_OUT_SPEC = pl.BlockSpec((ET, BLK, D), lambda i: (0, i, 0))
_Y_OUT_SHAPE = jax.ShapeDtypeStruct((ET, N, D), jnp.float32)


def _tc_y(h, wcat, be_r):
    def body(h_ref, wcat_ref, be_ref, y_ref):
        _y_blocks(h_ref[...], wcat_ref, be_ref, y_ref)

    return pl.pallas_call(
        body,
        grid=(GRID,),
        in_specs=[pl.BlockSpec((BLK, D), lambda i: (i, 0))] + _Y_SPECS,
        out_specs=_Y_OUT_SPEC,
        out_shape=_Y_OUT_SHAPE,
    )(h, wcat, be_r)


def _gru_block(parts_ref, h, wih_ref, whh_ref, bih_ref, bhh_ref):
    m = parts_ref[0] + parts_ref[1]
    gi = jnp.dot(m, wih_ref[...], preferred_element_type=jnp.float32) + bih_ref[...]
    gh = jnp.dot(h, whh_ref[...], preferred_element_type=jnp.float32) + bhh_ref[...]
    r = jax.nn.sigmoid(gi[:, :D] + gh[:, :D])
    z = jax.nn.sigmoid(gi[:, D:2 * D] + gh[:, D:2 * D])
    n = jnp.tanh(gi[:, 2 * D:] + r * gh[:, 2 * D:])
    return (1.0 - z) * n + z * h


_GRU_SPECS = [
    pl.BlockSpec((NC, BLK, D), lambda i: (0, i, 0)),
    pl.BlockSpec((BLK, D), lambda i: (i, 0)),
    pl.BlockSpec((D, 3 * D), lambda i: (0, 0)),
    pl.BlockSpec((D, 3 * D), lambda i: (0, 0)),
    pl.BlockSpec((1, 3 * D), lambda i: (0, 0)),
    pl.BlockSpec((1, 3 * D), lambda i: (0, 0)),
]


def _tc_gru_y(parts, h, wih_t, whh_t, bih_r, bhh_r, wcat, be_r):
    def body(parts_ref, h_ref, wih_ref, whh_ref, bih_ref, bhh_ref,
             wcat_ref, be_ref, h_out_ref, y_ref):
        hn = _gru_block(parts_ref, h_ref[...], wih_ref, whh_ref, bih_ref, bhh_ref)
        h_out_ref[...] = hn
        _y_blocks(hn, wcat_ref, be_ref, y_ref)

    return pl.pallas_call(
        body,
        grid=(GRID,),
        in_specs=_GRU_SPECS + _Y_SPECS,
        out_specs=[pl.BlockSpec((BLK, D), lambda i: (i, 0)), _Y_OUT_SPEC],
        out_shape=[
            jax.ShapeDtypeStruct((N, D), jnp.float32),
            _Y_OUT_SHAPE,
        ],
    )(parts, h, wih_t, whh_t, bih_r, bhh_r, wcat, be_r)


def _tc_final(parts, h, res, gids3, wih_t, whh_t, bih_r, bhh_r,
              w1t, b1_r, w2t, b2_r):
    def body(parts_ref, h_ref, res_ref, gid_ref, wih_ref, whh_ref,
             bih_ref, bhh_ref, w1t_ref, b1_ref, w2t_ref, b2_ref,
             out_ref, sums_ref, cnt_ref):
        i = pl.program_id(0)

        @pl.when(i == 0)
        def _():
            sums_ref[...] = jnp.zeros_like(sums_ref)
            cnt_ref[...] = jnp.zeros_like(cnt_ref)

        hn = _gru_block(parts_ref, h_ref[...], wih_ref, whh_ref, bih_ref, bhh_ref)

        ids = gid_ref[0, 0, :]
        oh = (
            lax.broadcasted_iota(jnp.int32, (B, BLK), 0) == ids[None, :]
        ).astype(jnp.float32)
        sums_ref[:, :D] += jnp.dot(oh, hn, preferred_element_type=jnp.float32)
        sums_ref[:, D:] += jnp.dot(
            oh, res_ref[...], preferred_element_type=jnp.float32
        )
        cnt_ref[...] += jnp.dot(
            oh, jnp.ones((BLK, D), jnp.float32), preferred_element_type=jnp.float32
        )

        @pl.when(i == pl.num_programs(0) - 1)
        def _():
            cnt = jnp.maximum(cnt_ref[...], 1.0)
            g = sums_ref[...] / jnp.concatenate([cnt] * ((D + RES) // D), axis=1)
            h1 = jax.nn.relu(
                jnp.dot(g, w1t_ref[...], preferred_element_type=jnp.float32)
                + b1_ref[...]
            )
            logit = (
                jnp.dot(h1, w2t_ref[...], preferred_element_type=jnp.float32)
                + b2_ref[...]
            )
            out_ref[...] = jax.nn.sigmoid(logit)

    return pl.pallas_call(
        body,
        grid=(GRID,),
        in_specs=_GRU_SPECS[:2] + [
            pl.BlockSpec((BLK, RES), lambda i: (i, 0)),
            pl.BlockSpec((1, 1, BLK), lambda i: (i, 0, 0)),
        ] + _GRU_SPECS[2:] + [
            pl.BlockSpec((D + RES, HID), lambda i: (0, 0)),
            pl.BlockSpec((1, HID), lambda i: (0, 0)),
            pl.BlockSpec((HID, 1), lambda i: (0, 0)),
            pl.BlockSpec((1, 1), lambda i: (0, 0)),
        ],
        out_specs=pl.BlockSpec((B, 1), lambda i: (0, 0)),
        out_shape=jax.ShapeDtypeStruct((B, 1), jnp.float32),
        scratch_shapes=[
            pltpu.VMEM((B, D + RES), jnp.float32),
            pltpu.VMEM((B, D), jnp.float32),
        ],
    )(parts, h, res, gids3, wih_t, whh_t, bih_r, bhh_r, w1t, b1_r, w2t, b2_r)


def kernel(features, edge_index, edge_types, graph_ids, We, be,
           W_ih, W_hh, b_ih, b_hh, W1, b1, W2, b2):
    x = features[:, :D]
    res = features[:, D:]

    src = edge_index[0].astype(jnp.int32)
    dst = edge_index[1].astype(jnp.int32)
    et = edge_types.astype(jnp.int32)
    pad = E_PAD - E
    gidx = et * N + src
    pad_g = jnp.arange(pad, dtype=jnp.int32) % (ET * N)
    gidx_p = jnp.concatenate([gidx, pad_g]).reshape(NW, CH, CSZ)
    pad_d = N + jnp.arange(pad, dtype=jnp.int32) % (NPAD - N)
    dst_p = jnp.concatenate([dst, pad_d]).reshape(NW, CH, CSZ)
    zrow = jnp.zeros((RPT, D), jnp.float32)

    wcat = jnp.transpose(We, (0, 2, 1))
    be_r = be[:, None, :]
    wih_t = W_ih.T
    whh_t = W_hh.T
    bih_r = b_ih[None, :]
    bhh_r = b_hh[None, :]
    w1t = W1.T
    b1_r = b1[None, :]
    w2t = W2.T
    b2_r = b2[None, :]
    gids3 = graph_ids.astype(jnp.int32).reshape(GRID, 1, BLK)

    h = x
    y = _tc_y(h, wcat, be_r)
    for step in range(STEPS):
        parts = _sc_aggregate(y.reshape(ET * N, D), gidx_p, dst_p, zrow)
        if step < STEPS - 1:
            h, y = _tc_gru_y(parts, h, wih_t, whh_t, bih_r, bhh_r, wcat, be_r)
        else:
            out = _tc_final(parts, h, res, gids3, wih_t, whh_t,
                            bih_r, bhh_r, w1t, b1_r, w2t, b2_r)
    return out


# speedup vs baseline: 1.2827x; 1.0012x over previous
import functools

import jax
import jax.numpy as jnp
from jax import lax
from jax.experimental import pallas as pl
from jax.experimental.pallas import tpu as pltpu
from jax.experimental.pallas import tpu_sc as plsc

N = 10000
E = 320000
D = 128
ET = 4
B = 16
STEPS = 8
HID = 256
RES = 768

NC = 2
NS = 16
NW = NC * NS
CSZ = 128
CH = 80
EPW = CH * CSZ
E_PAD = NW * EPW
NB = 2
RPT = 640
NPAD = NS * RPT
LAST = N - (NS - 1) * RPT


def _sc_aggregate(y2, gidx_p, dst_p, zrow):
    mesh = plsc.VectorSubcoreMesh(
        core_axis_name="c", subcore_axis_name="s", num_cores=NC, num_subcores=NS
    )

    @functools.partial(
        pl.kernel,
        out_type=jax.ShapeDtypeStruct((NC, N, D), jnp.float32),
        mesh=mesh,
        scratch_types=[
            pltpu.VMEM((CH, CSZ), jnp.int32),
            pltpu.VMEM((NB * 8, CSZ), jnp.int32),
            pltpu.VMEM((NB, CSZ, D), jnp.float32),
            pltpu.VMEM_SHARED((NPAD, D), jnp.float32),
            pltpu.SemaphoreType.DMA,
            pltpu.SemaphoreType.DMA,
            pltpu.SemaphoreType.DMA,
            pltpu.SemaphoreType.DMA,
        ],
    )
    def body(y_hbm, gidx_hbm, dst_hbm, z_hbm, out_hbm,
             gidx_v, didx_v, rows_v, m_sh, gs0, gs1, ds0, ds1):
        gsem = [gs0, gs1]
        dsem = [ds0, ds1]
        c = lax.axis_index("c")
        s = lax.axis_index("s")
        wid = c * NS + s

        dchunks = dst_hbm.at[wid]

        def issue_g(j, b):
            pltpu.async_copy(
                y_hbm.at[gidx_v.at[j]], rows_v.at[b], gsem[b]
            )

        def wait_g(j, b):
            pltpu.make_async_copy(
                y_hbm.at[gidx_v.at[j]], rows_v.at[b], gsem[b]
            ).wait()

        def issue_d(p, hb):
            pltpu.async_copy(dchunks.at[p], didx_v.at[pl.ds(hb * 8, NB)], dsem[hb])

        def wait_d(p, hb):
            pltpu.make_async_copy(
                dchunks.at[p], didx_v.at[pl.ds(hb * 8, NB)], dsem[hb]
            ).wait()

        pltpu.sync_copy(gidx_hbm.at[wid], gidx_v)
        for b in range(NB):
            issue_g(b, b)
        issue_d(0, 0)
        issue_d(1, 1)

        pltpu.sync_copy(z_hbm, m_sh.at[pl.ds(s * RPT, RPT)])
        plsc.subcore_barrier()

        @pl.loop(0, CH // NB, step=2)
        def _chunk(p0):
            for hb in range(2):
                p = p0 + hb
                wait_d(p, hb)
                for b in range(NB):
                    j = p * NB + b
                    wait_g(j, b)
                    pltpu.sync_copy(
                        rows_v.at[b], m_sh.at[didx_v.at[hb * 8 + b]], add=True
                    )

                    @pl.when(j + NB < CH)
                    def _():
                        issue_g(j + NB, b)

                @pl.when(p + 2 < CH // NB)
                def _():
                    issue_d(p + 2, hb)

        plsc.subcore_barrier()

        @pl.when(s < NS - 1)
        def _():
            pltpu.sync_copy(
                m_sh.at[pl.ds(s * RPT, RPT)], out_hbm.at[c].at[pl.ds(s * RPT, RPT)]
            )

        @pl.when(s == NS - 1)
        def _():
            pltpu.sync_copy(
                m_sh.at[pl.ds((NS - 1) * RPT, LAST)],
                out_hbm.at[c].at[pl.ds((NS - 1) * RPT, LAST)],
            )

    return body(y2, gidx_p, dst_p, zrow)


BLK = 1000
GRID = N // BLK


def _y_blocks(hn, wcat_ref, be_ref, y_ref):
    for t in range(ET):
        y_ref[t] = (
            jnp.dot(hn, wcat_ref[t], preferred_element_type=jnp.float32)
            + be_ref[t]
        )


_Y_SPECS = [
    pl.BlockSpec((ET, D, D), lambda i: (0, 0, 0)),
    pl.BlockSpec((ET, 1, D), lambda i: (0, 0, 0)),
]
_Y_OUT_SPEC = pl.BlockSpec((ET, BLK, D), lambda i: (0, i, 0))
_Y_OUT_SHAPE = jax.ShapeDtypeStruct((ET, N, D), jnp.float32)


def _tc_y(h, wcat, be_r):
    def body(h_ref, wcat_ref, be_ref, y_ref):
        _y_blocks(h_ref[...], wcat_ref, be_ref, y_ref)

    return pl.pallas_call(
        body,
        grid=(GRID,),
        in_specs=[pl.BlockSpec((BLK, D), lambda i: (i, 0))] + _Y_SPECS,
        out_specs=_Y_OUT_SPEC,
        out_shape=_Y_OUT_SHAPE,
    )(h, wcat, be_r)


def _gru_block(parts_ref, h, wih_ref, whh_ref, bih_ref, bhh_ref):
    m = parts_ref[0] + parts_ref[1]
    gi = jnp.dot(m, wih_ref[...], preferred_element_type=jnp.float32) + bih_ref[...]
    gh = jnp.dot(h, whh_ref[...], preferred_element_type=jnp.float32) + bhh_ref[...]
    r = jax.nn.sigmoid(gi[:, :D] + gh[:, :D])
    z = jax.nn.sigmoid(gi[:, D:2 * D] + gh[:, D:2 * D])
    n = jnp.tanh(gi[:, 2 * D:] + r * gh[:, 2 * D:])
    return (1.0 - z) * n + z * h


_GRU_SPECS = [
    pl.BlockSpec((NC, BLK, D), lambda i: (0, i, 0)),
    pl.BlockSpec((BLK, D), lambda i: (i, 0)),
    pl.BlockSpec((D, 3 * D), lambda i: (0, 0)),
    pl.BlockSpec((D, 3 * D), lambda i: (0, 0)),
    pl.BlockSpec((1, 3 * D), lambda i: (0, 0)),
    pl.BlockSpec((1, 3 * D), lambda i: (0, 0)),
]


def _tc_gru_y(parts, h, wih_t, whh_t, bih_r, bhh_r, wcat, be_r):
    def body(parts_ref, h_ref, wih_ref, whh_ref, bih_ref, bhh_ref,
             wcat_ref, be_ref, h_out_ref, y_ref):
        hn = _gru_block(parts_ref, h_ref[...], wih_ref, whh_ref, bih_ref, bhh_ref)
        h_out_ref[...] = hn
        _y_blocks(hn, wcat_ref, be_ref, y_ref)

    return pl.pallas_call(
        body,
        grid=(GRID,),
        in_specs=_GRU_SPECS + _Y_SPECS,
        out_specs=[pl.BlockSpec((BLK, D), lambda i: (i, 0)), _Y_OUT_SPEC],
        out_shape=[
            jax.ShapeDtypeStruct((N, D), jnp.float32),
            _Y_OUT_SHAPE,
        ],
    )(parts, h, wih_t, whh_t, bih_r, bhh_r, wcat, be_r)


def _tc_final(parts, h, res, gids3, wih_t, whh_t, bih_r, bhh_r,
              w1t, b1_r, w2t, b2_r):
    def body(parts_ref, h_ref, res_ref, gid_ref, wih_ref, whh_ref,
             bih_ref, bhh_ref, w1t_ref, b1_ref, w2t_ref, b2_ref,
             out_ref, sums_ref, cnt_ref):
        i = pl.program_id(0)

        @pl.when(i == 0)
        def _():
            sums_ref[...] = jnp.zeros_like(sums_ref)
            cnt_ref[...] = jnp.zeros_like(cnt_ref)

        hn = _gru_block(parts_ref, h_ref[...], wih_ref, whh_ref, bih_ref, bhh_ref)

        ids = gid_ref[0, 0, :]
        oh = (
            lax.broadcasted_iota(jnp.int32, (B, BLK), 0) == ids[None, :]
        ).astype(jnp.float32)
        sums_ref[:, :D] += jnp.dot(oh, hn, preferred_element_type=jnp.float32)
        sums_ref[:, D:] += jnp.dot(
            oh, res_ref[...], preferred_element_type=jnp.float32
        )
        cnt_ref[...] += jnp.dot(
            oh, jnp.ones((BLK, D), jnp.float32), preferred_element_type=jnp.float32
        )

        @pl.when(i == pl.num_programs(0) - 1)
        def _():
            cnt = jnp.maximum(cnt_ref[...], 1.0)
            g = sums_ref[...] / jnp.concatenate([cnt] * ((D + RES) // D), axis=1)
            h1 = jax.nn.relu(
                jnp.dot(g, w1t_ref[...], preferred_element_type=jnp.float32)
                + b1_ref[...]
            )
            logit = (
                jnp.dot(h1, w2t_ref[...], preferred_element_type=jnp.float32)
                + b2_ref[...]
            )
            out_ref[...] = jax.nn.sigmoid(logit)

    return pl.pallas_call(
        body,
        grid=(GRID,),
        in_specs=_GRU_SPECS[:2] + [
            pl.BlockSpec((BLK, RES), lambda i: (i, 0)),
            pl.BlockSpec((1, 1, BLK), lambda i: (i, 0, 0)),
        ] + _GRU_SPECS[2:] + [
            pl.BlockSpec((D + RES, HID), lambda i: (0, 0)),
            pl.BlockSpec((1, HID), lambda i: (0, 0)),
            pl.BlockSpec((HID, 1), lambda i: (0, 0)),
            pl.BlockSpec((1, 1), lambda i: (0, 0)),
        ],
        out_specs=pl.BlockSpec((B, 1), lambda i: (0, 0)),
        out_shape=jax.ShapeDtypeStruct((B, 1), jnp.float32),
        scratch_shapes=[
            pltpu.VMEM((B, D + RES), jnp.float32),
            pltpu.VMEM((B, D), jnp.float32),
        ],
    )(parts, h, res, gids3, wih_t, whh_t, bih_r, bhh_r, w1t, b1_r, w2t, b2_r)


def kernel(features, edge_index, edge_types, graph_ids, We, be,
           W_ih, W_hh, b_ih, b_hh, W1, b1, W2, b2):
    x = features[:, :D]
    res = features[:, D:]

    src = edge_index[0].astype(jnp.int32)
    dst = edge_index[1].astype(jnp.int32)
    et = edge_types.astype(jnp.int32)
    pad = E_PAD - E
    gidx = et * N + src
    pad_g = jnp.arange(pad, dtype=jnp.int32) % (ET * N)
    gidx_p = jnp.concatenate([gidx, pad_g]).reshape(NW, CH, CSZ)
    pad_d = N + jnp.arange(pad, dtype=jnp.int32) % (NPAD - N)
    dst_p = jnp.concatenate([dst, pad_d]).reshape(NW, CH // NB, NB, CSZ)
    zrow = jnp.zeros((RPT, D), jnp.float32)

    wcat = jnp.transpose(We, (0, 2, 1))
    be_r = be[:, None, :]
    wih_t = W_ih.T
    whh_t = W_hh.T
    bih_r = b_ih[None, :]
    bhh_r = b_hh[None, :]
    w1t = W1.T
    b1_r = b1[None, :]
    w2t = W2.T
    b2_r = b2[None, :]
    gids3 = graph_ids.astype(jnp.int32).reshape(GRID, 1, BLK)

    h = x
    y = _tc_y(h, wcat, be_r)
    for step in range(STEPS):
        parts = _sc_aggregate(y.reshape(ET * N, D), gidx_p, dst_p, zrow)
        if step < STEPS - 1:
            h, y = _tc_gru_y(parts, h, wih_t, whh_t, bih_r, bhh_r, wcat, be_r)
        else:
            out = _tc_final(parts, h, res, gids3, wih_t, whh_t,
                            bih_r, bhh_r, w1t, b1_r, w2t, b2_r)
    return out
